# baseline (speedup 1.0000x reference)
import functools

import jax
import jax.numpy as jnp
from jax import lax
from jax.experimental import pallas as pl
from jax.experimental.pallas import tpu as pltpu
from jax.experimental.pallas import tpu_sc as plsc

N_FACTORS = 64
BATCH = 16384
N_ROWS = 1000000
D2 = 128
V2 = N_ROWS // 2
IDX_CHUNK = 128
CHUNK = 256

_info = plsc.get_sparse_core_info()
_NC, _NS, _L = _info.num_cores, _info.num_subcores, _info.num_lanes
_NW = _NC * _NS
_B_PER_W = BATCH // _NW
_N_CHUNKS = _B_PER_W // CHUNK
_G_PER_CHUNK = CHUNK // _L


def _bpr_body(x_flat, user_t, item_t, out, idx_a, idx_g, idx_o,
              gu_b, gi_b, gj_b, out_v, sem):
    wid = lax.axis_index("s") * _NC + lax.axis_index("c")
    base = wid * _B_PER_W

    pltpu.sync_copy(x_flat.at[pl.ds(base, _B_PER_W)],
                    idx_a.at[pl.ds(0, _B_PER_W)])
    pltpu.sync_copy(x_flat.at[pl.ds(BATCH + base, _B_PER_W)],
                    idx_a.at[pl.ds(_B_PER_W, _B_PER_W)])
    pltpu.sync_copy(x_flat.at[pl.ds(2 * BATCH + base, _B_PER_W)],
                    idx_a.at[pl.ds(2 * _B_PER_W, _B_PER_W)])

    def split_body(i, c):
        sl = pl.ds(i * _L, _L)
        v = idx_a[sl]
        idx_g[sl] = lax.shift_right_logical(v, 1)
        idx_o[sl] = lax.shift_left(lax.bitwise_and(v, 1), 6)
        return c

    lax.fori_loop(0, (3 * _B_PER_W) // _L, split_body, 0)

    lanes = lax.iota(jnp.int32, _L)

    def chunk_body(c, loss_acc):
        cbase = c * CHUNK
        copies = []
        for h in range(CHUNK // IDX_CHUNK):
            dst = pl.ds(h * IDX_CHUNK, IDX_CHUNK)
            src_u = idx_g.at[pl.ds(cbase + h * IDX_CHUNK, IDX_CHUNK)]
            src_i = idx_g.at[pl.ds(_B_PER_W + cbase + h * IDX_CHUNK, IDX_CHUNK)]
            src_j = idx_g.at[pl.ds(2 * _B_PER_W + cbase + h * IDX_CHUNK, IDX_CHUNK)]
            copies.append(pltpu.async_copy(user_t.at[src_u], gu_b.at[dst], sem))
            copies.append(pltpu.async_copy(item_t.at[src_i], gi_b.at[dst], sem))
            copies.append(pltpu.async_copy(item_t.at[src_j], gj_b.at[dst], sem))
        for cp in copies:
            cp.wait()

        def group_body(g, acc_in):
            slot = g * _L + lanes
            ou = idx_o[pl.ds(cbase + g * _L, _L)]
            oi = idx_o[pl.ds(_B_PER_W + cbase + g * _L, _L)]
            oj = idx_o[pl.ds(2 * _B_PER_W + cbase + g * _L, _L)]

            def col_body(k, acc):
                kvec = jnp.zeros((_L,), jnp.int32) + k
                gu = plsc.load_gather(gu_b, [slot, ou + kvec])
                gi = plsc.load_gather(gi_b, [slot, oi + kvec])
                gj = plsc.load_gather(gj_b, [slot, oj + kvec])
                return acc + gu * (gi - gj)

            z = lax.fori_loop(0, N_FACTORS, col_body,
                              jnp.zeros((_L,), jnp.float32))
            e = jnp.exp(-jnp.abs(z))
            w = e / (2.0 + e)
            w2 = w * w
            ln1p = 2.0 * w * (1.0 + w2 * (
                (1.0 / 3.0) + w2 * ((1.0 / 5.0) + w2 * (
                    (1.0 / 7.0) + w2 * (1.0 / 9.0)))))
            return acc_in + jnp.maximum(-z, 0.0) + ln1p

        return lax.fori_loop(0, _G_PER_CHUNK, group_body, loss_acc)

    loss_acc = lax.fori_loop(0, _N_CHUNKS, chunk_body,
                             jnp.zeros((_L,), jnp.float32))
    total = jnp.sum(loss_acc)

    def out_fill(q, c):
        out_v[pl.ds(q * _L, _L)] = jnp.zeros((_L,), jnp.float32) + total
        return c

    lax.fori_loop(0, D2 // _L, out_fill, 0)
    pltpu.sync_copy(out_v, out.at[pl.ds(wid * D2, D2)])


_bpr_sc = functools.partial(
    pl.kernel,
    mesh=plsc.VectorSubcoreMesh(core_axis_name="c", subcore_axis_name="s"),
    compiler_params=pltpu.CompilerParams(needs_layout_passes=False),
    out_type=jax.ShapeDtypeStruct((_NW * D2,), jnp.float32),
    scratch_types=[
        pltpu.VMEM((3 * _B_PER_W,), jnp.int32),
        pltpu.VMEM((3 * _B_PER_W,), jnp.int32),
        pltpu.VMEM((3 * _B_PER_W,), jnp.int32),
        pltpu.VMEM((CHUNK, D2), jnp.float32),
        pltpu.VMEM((CHUNK, D2), jnp.float32),
        pltpu.VMEM((CHUNK, D2), jnp.float32),
        pltpu.VMEM((D2,), jnp.float32),
        pltpu.SemaphoreType.DMA,
    ],
)(_bpr_body)


def _pair_rows(t):
    return jnp.concatenate([t[0::2], t[1::2]], axis=1)


def kernel(x, user_embeddings, item_embeddings):
    x_flat = x.astype(jnp.int32).reshape(3 * BATCH)
    user_t = _pair_rows(user_embeddings)
    item_t = _pair_rows(item_embeddings)
    partials = _bpr_sc(x_flat, user_t, item_t)
    return jnp.sum(partials.reshape(_NW, D2)[:, 0])


# speedup vs baseline: 15.1412x; 15.1412x over previous
import functools

import jax
import jax.numpy as jnp
from jax import lax
from jax.experimental import pallas as pl
from jax.experimental.pallas import tpu as pltpu
from jax.experimental.pallas import tpu_sc as plsc

N_FACTORS = 64
BATCH = 16384
N_ROWS = 1000000
D2 = 128
V2 = N_ROWS // 2
IDX_CHUNK = 128
CHUNK = 256

_info = plsc.get_sparse_core_info()
_NC, _NS, _L = _info.num_cores, _info.num_subcores, _info.num_lanes
_NW = _NC * _NS
_B_PER_W = BATCH // _NW
_N_CHUNKS = _B_PER_W // CHUNK
_G_PER_CHUNK = CHUNK // _L


def _bpr_body(x_flat, user_t, item_t, out, idx_a, idx_g, idx_o,
              gu_b, gi_b, gj_b, out_v, sem):
    wid = lax.axis_index("s") * _NC + lax.axis_index("c")
    base = wid * _B_PER_W

    pltpu.sync_copy(x_flat.at[pl.ds(base, _B_PER_W)],
                    idx_a.at[pl.ds(0, _B_PER_W)])
    pltpu.sync_copy(x_flat.at[pl.ds(BATCH + base, _B_PER_W)],
                    idx_a.at[pl.ds(_B_PER_W, _B_PER_W)])
    pltpu.sync_copy(x_flat.at[pl.ds(2 * BATCH + base, _B_PER_W)],
                    idx_a.at[pl.ds(2 * _B_PER_W, _B_PER_W)])

    def split_body(i, c):
        sl = pl.ds(i * _L, _L)
        v = idx_a[sl]
        idx_g[sl] = lax.shift_right_logical(v, 1)
        idx_o[sl] = lax.shift_left(lax.bitwise_and(v, 1), 6)
        return c

    lax.fori_loop(0, (3 * _B_PER_W) // _L, split_body, 0)

    lanes = lax.iota(jnp.int32, _L)

    def chunk_body(c, loss_acc):
        cbase = c * CHUNK
        copies = []
        for h in range(CHUNK // IDX_CHUNK):
            dst = pl.ds(h * IDX_CHUNK, IDX_CHUNK)
            src_u = idx_g.at[pl.ds(cbase + h * IDX_CHUNK, IDX_CHUNK)]
            src_i = idx_g.at[pl.ds(_B_PER_W + cbase + h * IDX_CHUNK, IDX_CHUNK)]
            src_j = idx_g.at[pl.ds(2 * _B_PER_W + cbase + h * IDX_CHUNK, IDX_CHUNK)]
            copies.append(pltpu.async_copy(user_t.at[src_u], gu_b.at[dst], sem))
            copies.append(pltpu.async_copy(item_t.at[src_i], gi_b.at[dst], sem))
            copies.append(pltpu.async_copy(item_t.at[src_j], gj_b.at[dst], sem))
        for cp in copies:
            cp.wait()

        def group_body(g, acc_in):
            slot = g * _L + lanes
            ou = idx_o[pl.ds(cbase + g * _L, _L)]
            oi = idx_o[pl.ds(_B_PER_W + cbase + g * _L, _L)]
            oj = idx_o[pl.ds(2 * _B_PER_W + cbase + g * _L, _L)]

            def col_body(k, acc):
                kvec = jnp.zeros((_L,), jnp.int32) + k
                gu = plsc.load_gather(gu_b, [slot, ou + kvec])
                gi = plsc.load_gather(gi_b, [slot, oi + kvec])
                gj = plsc.load_gather(gj_b, [slot, oj + kvec])
                return acc + gu * (gi - gj)

            z = lax.fori_loop(0, N_FACTORS, col_body,
                              jnp.zeros((_L,), jnp.float32))
            e = jnp.exp(-jnp.abs(z))
            w = e / (2.0 + e)
            w2 = w * w
            ln1p = 2.0 * w * (1.0 + w2 * (
                (1.0 / 3.0) + w2 * ((1.0 / 5.0) + w2 * (
                    (1.0 / 7.0) + w2 * (1.0 / 9.0)))))
            return acc_in + jnp.maximum(-z, 0.0) + ln1p

        return lax.fori_loop(0, _G_PER_CHUNK, group_body, loss_acc)

    loss_acc = lax.fori_loop(0, _N_CHUNKS, chunk_body,
                             jnp.zeros((_L,), jnp.float32))
    total = jnp.sum(loss_acc)

    def out_fill(q, c):
        out_v[pl.ds(q * _L, _L)] = jnp.zeros((_L,), jnp.float32) + total
        return c

    lax.fori_loop(0, D2 // _L, out_fill, 0)
    pltpu.sync_copy(out_v, out.at[pl.ds(wid * D2, D2)])


_bpr_sc = functools.partial(
    pl.kernel,
    mesh=plsc.VectorSubcoreMesh(core_axis_name="c", subcore_axis_name="s"),
    compiler_params=pltpu.CompilerParams(
        needs_layout_passes=False, skip_device_barrier=True),
    out_type=jax.ShapeDtypeStruct((_NW * D2,), jnp.float32),
    scratch_types=[
        pltpu.VMEM((3 * _B_PER_W,), jnp.int32),
        pltpu.VMEM((3 * _B_PER_W,), jnp.int32),
        pltpu.VMEM((3 * _B_PER_W,), jnp.int32),
        pltpu.VMEM((CHUNK, D2), jnp.float32),
        pltpu.VMEM((CHUNK, D2), jnp.float32),
        pltpu.VMEM((CHUNK, D2), jnp.float32),
        pltpu.VMEM((D2,), jnp.float32),
        pltpu.SemaphoreType.DMA,
    ],
)(_bpr_body)


def kernel(x, user_embeddings, item_embeddings):
    x_flat = x.astype(jnp.int32).reshape(3 * BATCH)
    user_t = user_embeddings.reshape(V2, D2)
    item_t = item_embeddings.reshape(V2, D2)
    partials = _bpr_sc(x_flat, user_t, item_t)
    return jnp.sum(partials.reshape(_NW, D2)[:, 0])


# speedup vs baseline: 15.2071x; 1.0044x over previous
import functools

import jax
import jax.numpy as jnp
from jax import lax
from jax.experimental import pallas as pl
from jax.experimental.pallas import tpu as pltpu
from jax.experimental.pallas import tpu_sc as plsc

N_FACTORS = 64
BATCH = 16384
IDX_CHUNK = 128

_info = plsc.get_sparse_core_info()
_NC, _NS, _L = _info.num_cores, _info.num_subcores, _info.num_lanes
_NW = _NC * _NS
_B_PER_W = BATCH // _NW
_N_CHUNKS = _B_PER_W // IDX_CHUNK
_N_GROUPS = _B_PER_W // _L


def _bpr_body(xr, user_t, item_t, out, idx_v, ru, ri, rj, out_v, sem):
    wid = lax.axis_index("s") * _NC + lax.axis_index("c")

    pltpu.sync_copy(xr.at[0, pl.ds(wid * _N_CHUNKS, _N_CHUNKS)], idx_v.at[0])
    pltpu.sync_copy(xr.at[1, pl.ds(wid * _N_CHUNKS, _N_CHUNKS)], idx_v.at[1])
    pltpu.sync_copy(xr.at[2, pl.ds(wid * _N_CHUNKS, _N_CHUNKS)], idx_v.at[2])

    copies = []
    for j in range(_N_CHUNKS):
        dst = pl.ds(j * IDX_CHUNK, IDX_CHUNK)
        copies.append(pltpu.async_copy(user_t.at[idx_v.at[0, j]], ru.at[dst], sem))
        copies.append(pltpu.async_copy(item_t.at[idx_v.at[1, j]], ri.at[dst], sem))
        copies.append(pltpu.async_copy(item_t.at[idx_v.at[2, j]], rj.at[dst], sem))
    for c in copies:
        c.wait()

    lanes = lax.iota(jnp.int32, _L)

    def group_body(g, loss_acc):
        ridx = g * _L + lanes

        def col_body(k, acc):
            cidx = jnp.zeros((_L,), jnp.int32) + k
            gu = plsc.load_gather(ru, [ridx, cidx])
            gi = plsc.load_gather(ri, [ridx, cidx])
            gj = plsc.load_gather(rj, [ridx, cidx])
            return acc + gu * (gi - gj)

        z = lax.fori_loop(0, N_FACTORS, col_body, jnp.zeros((_L,), jnp.float32))
        e = jnp.exp(-jnp.abs(z))
        w = e / (2.0 + e)
        w2 = w * w
        ln1p = 2.0 * w * (1.0 + w2 * (
            (1.0 / 3.0) + w2 * ((1.0 / 5.0) + w2 * ((1.0 / 7.0) + w2 * (1.0 / 9.0)))))
        return loss_acc + jnp.maximum(-z, 0.0) + ln1p

    loss_acc = lax.fori_loop(0, _N_GROUPS, group_body, jnp.zeros((_L,), jnp.float32))
    total = jnp.sum(loss_acc)
    out_v[...] = jnp.zeros((_L,), jnp.float32) + total
    pltpu.sync_copy(out_v, out.at[pl.ds(wid * _L, _L)])


_bpr_sc = functools.partial(
    pl.kernel,
    mesh=plsc.VectorSubcoreMesh(core_axis_name="c", subcore_axis_name="s"),
    compiler_params=pltpu.CompilerParams(
        needs_layout_passes=False, use_tc_tiling_on_sc=False),
    out_type=jax.ShapeDtypeStruct((_NW * _L,), jnp.float32),
    scratch_types=[
        pltpu.VMEM((3, _N_CHUNKS, IDX_CHUNK), jnp.int32),
        pltpu.VMEM((_B_PER_W, N_FACTORS), jnp.float32),
        pltpu.VMEM((_B_PER_W, N_FACTORS), jnp.float32),
        pltpu.VMEM((_B_PER_W, N_FACTORS), jnp.float32),
        pltpu.VMEM((_L,), jnp.float32),
        pltpu.SemaphoreType.DMA,
    ],
)(_bpr_body)


def kernel(x, user_embeddings, item_embeddings):
    xr = x.astype(jnp.int32).reshape(3, BATCH // IDX_CHUNK, IDX_CHUNK)
    partials = _bpr_sc(xr, user_embeddings, item_embeddings)
    return jnp.sum(partials.reshape(_NW, _L)[:, 0])


# speedup vs baseline: 15.2136x; 1.0004x over previous
import functools

import jax
import jax.numpy as jnp
from jax import lax
from jax.experimental import pallas as pl
from jax.experimental.pallas import tpu as pltpu
from jax.experimental.pallas import tpu_sc as plsc

N_FACTORS = 64
BATCH = 16384
IDX_CHUNK = 128

_info = plsc.get_sparse_core_info()
_NC, _NS, _L = _info.num_cores, _info.num_subcores, _info.num_lanes
_NW = _NC * _NS
_B_PER_W = BATCH // _NW
_N_CHUNKS = _B_PER_W // IDX_CHUNK
_N_GROUPS = _B_PER_W // _L


def _bpr_body(xr, user_t, item_t, out, idx_v, ru, ri, rj, out_v, sem):
    wid = lax.axis_index("s") * _NC + lax.axis_index("c")

    pltpu.sync_copy(xr.at[0, pl.ds(wid * _N_CHUNKS, _N_CHUNKS)], idx_v.at[0])
    pltpu.sync_copy(xr.at[1, pl.ds(wid * _N_CHUNKS, _N_CHUNKS)], idx_v.at[1])
    pltpu.sync_copy(xr.at[2, pl.ds(wid * _N_CHUNKS, _N_CHUNKS)], idx_v.at[2])

    copies = []
    for j in range(_N_CHUNKS):
        dst = pl.ds(j * IDX_CHUNK, IDX_CHUNK)
        copies.append(pltpu.async_copy(user_t.at[idx_v.at[0, j]], ru.at[dst], sem))
        copies.append(pltpu.async_copy(item_t.at[idx_v.at[1, j]], ri.at[dst], sem))
        copies.append(pltpu.async_copy(item_t.at[idx_v.at[2, j]], rj.at[dst], sem))
    for c in copies:
        c.wait()

    lanes = lax.iota(jnp.int32, _L)

    def group_body(g, loss_acc):
        ridx = g * _L + lanes

        def col_body(k, acc):
            cidx = jnp.zeros((_L,), jnp.int32) + k
            gu = plsc.load_gather(ru, [ridx, cidx])
            gi = plsc.load_gather(ri, [ridx, cidx])
            gj = plsc.load_gather(rj, [ridx, cidx])
            return acc + gu * (gi - gj)

        z = lax.fori_loop(0, N_FACTORS, col_body, jnp.zeros((_L,), jnp.float32))
        e = jnp.exp(-jnp.abs(z))
        w = e / (2.0 + e)
        w2 = w * w
        ln1p = 2.0 * w * (1.0 + w2 * (
            (1.0 / 3.0) + w2 * ((1.0 / 5.0) + w2 * ((1.0 / 7.0) + w2 * (1.0 / 9.0)))))
        return loss_acc + jnp.maximum(-z, 0.0) + ln1p

    loss_acc = lax.fori_loop(0, _N_GROUPS, group_body, jnp.zeros((_L,), jnp.float32))
    total = jnp.sum(loss_acc)
    out_v[...] = jnp.zeros((_L,), jnp.float32) + total
    pltpu.sync_copy(out_v, out.at[pl.ds(wid * _L, _L)])


_bpr_sc = functools.partial(
    pl.kernel,
    mesh=plsc.VectorSubcoreMesh(core_axis_name="c", subcore_axis_name="s"),
    compiler_params=pltpu.CompilerParams(
        needs_layout_passes=False, use_tc_tiling_on_sc=False),
    cost_estimate=pl.CostEstimate(
        flops=3 * BATCH * N_FACTORS * 2,
        transcendentals=BATCH,
        bytes_accessed=3 * BATCH * N_FACTORS * 4),
    out_type=jax.ShapeDtypeStruct((_NW * _L,), jnp.float32),
    scratch_types=[
        pltpu.VMEM((3, _N_CHUNKS, IDX_CHUNK), jnp.int32),
        pltpu.VMEM((_B_PER_W, N_FACTORS), jnp.float32),
        pltpu.VMEM((_B_PER_W, N_FACTORS), jnp.float32),
        pltpu.VMEM((_B_PER_W, N_FACTORS), jnp.float32),
        pltpu.VMEM((_L,), jnp.float32),
        pltpu.SemaphoreType.DMA,
    ],
)(_bpr_body)


def kernel(x, user_embeddings, item_embeddings):
    xr = x.astype(jnp.int32).reshape(3, BATCH // IDX_CHUNK, IDX_CHUNK)
    partials = _bpr_sc(xr, user_embeddings, item_embeddings)
    return jnp.sum(partials.reshape(_NW, _L)[:, 0])
